# Initial kernel scaffold; baseline (speedup 1.0000x reference)
import jax
import jax.numpy as jnp
from jax.experimental import pallas as pl


def kernel(inputs, support0, support1, W_ru_e, b_ru_e, W_c_e, b_c_e, W_ru_d, b_ru_d, W_c_d, b_c_d, W_proj, b_proj):
    raise NotImplementedError("write your pallas kernel here")


# speedup vs baseline: 5.7341x; 5.7341x over previous
import jax
import jax.numpy as jnp
from jax.experimental import pallas as pl

N = 207
B = 16
L = 8
HORIZON = 8
U = 64
IN_F = 72
NUM_MAT = 5
F32 = jnp.float32


def _fwd_kernel(x_all_ref, s0t_ref, s1t_ref,
                w_ru_e_ref, b_ru_e_ref, w_c_e_ref, b_c_e_ref,
                w_ru_d_ref, b_ru_d_ref, w_c_d_ref, b_c_d_ref,
                w_projt_ref, b_proj_ref,
                out_ref):
    s0t = s0t_ref[...]
    s1t = s1t_ref[...]

    def to_diff(v):
        return jnp.transpose(v.reshape(B, N, IN_F), (0, 2, 1)).reshape(B * IN_F, N)

    def from_diff(v):
        return jnp.transpose(v.reshape(B, IN_F, N), (0, 2, 1)).reshape(B * N, IN_F)

    def gconv(cat, w_re, b):
        z0 = to_diff(cat)
        z1a = jnp.dot(z0, s0t, preferred_element_type=F32)
        z2a = 2.0 * jnp.dot(z1a, s0t, preferred_element_type=F32) - z0
        z1b = jnp.dot(z0, s1t, preferred_element_type=F32)
        z2b = 2.0 * jnp.dot(z1b, s1t, preferred_element_type=F32) - z0
        xall = jnp.concatenate(
            [cat] + [from_diff(z) for z in (z1a, z2a, z1b, z2b)], axis=1)
        return jnp.dot(xall, w_re, preferred_element_type=F32) + b

    def cell(x, h, w_ru, b_ru, w_c, b_c, pad):
        def mkcat(state):
            parts = [x, state]
            if pad:
                parts.append(jnp.zeros((B * N, pad), F32))
            return jnp.concatenate(parts, axis=1)
        ru = jax.nn.sigmoid(gconv(mkcat(h), w_ru, b_ru))
        r = ru[:, :U]
        u = ru[:, U:]
        c = jnp.tanh(gconv(mkcat(r * h), w_c, b_c))
        return u * h + (1.0 - u) * c

    w_ru_e = w_ru_e_ref[...]
    b_ru_e = b_ru_e_ref[...]
    w_c_e = w_c_e_ref[...]
    b_c_e = b_c_e_ref[...]

    def enc_body(t, h):
        x = x_all_ref[pl.ds(t, 1)].reshape(B * N, L)
        return cell(x, h, w_ru_e, b_ru_e, w_c_e, b_c_e, 0)

    h = jax.lax.fori_loop(0, L, enc_body, jnp.zeros((B * N, U), F32))

    w_ru_d = w_ru_d_ref[...]
    b_ru_d = b_ru_d_ref[...]
    w_c_d = w_c_d_ref[...]
    b_c_d = b_c_d_ref[...]
    w_projt = w_projt_ref[...]
    b_proj = b_proj_ref[0, 0]

    def dec_body(t, h):
        projt = jnp.dot(w_projt, h.T, preferred_element_type=F32) + b_proj
        xin = jnp.where(t == 0, jnp.zeros_like(projt), projt).T
        h2 = cell(xin, h, w_ru_d, b_ru_d, w_c_d, b_c_d, IN_F - 1 - U)
        proj2 = jnp.dot(w_projt, h2.T, preferred_element_type=F32) + b_proj
        out_ref[pl.ds(t, 1)] = proj2
        return h2

    jax.lax.fori_loop(0, HORIZON, dec_body, h)


def _reorder_w(w, din, dout):
    wr = w.reshape(din, NUM_MAT, dout).transpose(1, 0, 2)
    if din < IN_F:
        wr = jnp.pad(wr, ((0, 0), (0, IN_F - din), (0, 0)))
    return wr.reshape(NUM_MAT * IN_F, dout)


def kernel(inputs, support0, support1, W_ru_e, b_ru_e, W_c_e, b_c_e,
           W_ru_d, b_ru_d, W_c_d, b_c_d, W_proj, b_proj):
    x_all = inputs.reshape(L, B * N, L)

    out = pl.pallas_call(
        _fwd_kernel,
        out_shape=jax.ShapeDtypeStruct((HORIZON, B * N), F32),
    )(x_all, support0.T, support1.T,
      _reorder_w(W_ru_e, L + U, 2 * U), b_ru_e.reshape(1, 2 * U),
      _reorder_w(W_c_e, L + U, U), b_c_e.reshape(1, U),
      _reorder_w(W_ru_d, 1 + U, 2 * U), b_ru_d.reshape(1, 2 * U),
      _reorder_w(W_c_d, 1 + U, U), b_c_d.reshape(1, U),
      W_proj.T, b_proj.reshape(1, 1))

    return out.reshape(HORIZON, B, N)


# speedup vs baseline: 7.8733x; 1.3731x over previous
import jax
import jax.numpy as jnp
from jax.experimental import pallas as pl

N = 207
B = 16
L = 8
HORIZON = 8
U = 64
IN_F = 72
NUM_MAT = 5
F32 = jnp.float32
BF16 = jnp.bfloat16


def _fwd_kernel(x_all_ref, s0t_ref, s1t_ref,
                w_ru_e_ref, b_ru_e_ref, w_c_e_ref, b_c_e_ref,
                w_ru_d_ref, b_ru_d_ref, w_c_d_ref, b_c_d_ref,
                w_projt_ref, b_proj_ref,
                out_ref):
    s0t = s0t_ref[...]
    s1t = s1t_ref[...]

    def to_diff(v):
        return jnp.transpose(v.reshape(B, N, IN_F), (0, 2, 1)).reshape(B * IN_F, N)

    def from_diff(v):
        return jnp.transpose(v.reshape(B, IN_F, N), (0, 2, 1)).reshape(B * N, IN_F)

    def gconv(cat, w_re, b):
        catb = cat.astype(BF16)
        z0 = to_diff(catb)
        z1a = jnp.dot(z0, s0t, preferred_element_type=F32).astype(BF16)
        z2a = (2.0 * jnp.dot(z1a, s0t, preferred_element_type=F32)
               - z0.astype(F32)).astype(BF16)
        z1b = jnp.dot(z0, s1t, preferred_element_type=F32).astype(BF16)
        z2b = (2.0 * jnp.dot(z1b, s1t, preferred_element_type=F32)
               - z0.astype(F32)).astype(BF16)
        xall = jnp.concatenate(
            [catb] + [from_diff(z) for z in (z1a, z2a, z1b, z2b)], axis=1)
        return jnp.dot(xall, w_re, preferred_element_type=F32) + b

    def cell(x, h, w_ru, b_ru, w_c, b_c, pad):
        def mkcat(state):
            parts = [x, state]
            if pad:
                parts.append(jnp.zeros((B * N, pad), F32))
            return jnp.concatenate(parts, axis=1)
        ru = jax.nn.sigmoid(gconv(mkcat(h), w_ru, b_ru))
        r = ru[:, :U]
        u = ru[:, U:]
        c = jnp.tanh(gconv(mkcat(r * h), w_c, b_c))
        return u * h + (1.0 - u) * c

    w_ru_e = w_ru_e_ref[...]
    b_ru_e = b_ru_e_ref[...]
    w_c_e = w_c_e_ref[...]
    b_c_e = b_c_e_ref[...]

    def enc_body(t, h):
        x = x_all_ref[pl.ds(t, 1)].reshape(B * N, L)
        return cell(x, h, w_ru_e, b_ru_e, w_c_e, b_c_e, 0)

    h = jax.lax.fori_loop(0, L, enc_body, jnp.zeros((B * N, U), F32))

    w_ru_d = w_ru_d_ref[...]
    b_ru_d = b_ru_d_ref[...]
    w_c_d = w_c_d_ref[...]
    b_c_d = b_c_d_ref[...]
    w_projt = w_projt_ref[...]
    b_proj = b_proj_ref[0, 0]

    def dec_body(t, h):
        projt = jnp.dot(w_projt, h.T, preferred_element_type=F32) + b_proj
        xin = jnp.where(t == 0, jnp.zeros_like(projt), projt).T
        h2 = cell(xin, h, w_ru_d, b_ru_d, w_c_d, b_c_d, IN_F - 1 - U)
        proj2 = jnp.dot(w_projt, h2.T, preferred_element_type=F32) + b_proj
        out_ref[pl.ds(t, 1)] = proj2
        return h2

    jax.lax.fori_loop(0, HORIZON, dec_body, h)


def _reorder_w(w, din, dout):
    wr = w.reshape(din, NUM_MAT, dout).transpose(1, 0, 2)
    if din < IN_F:
        wr = jnp.pad(wr, ((0, 0), (0, IN_F - din), (0, 0)))
    return wr.reshape(NUM_MAT * IN_F, dout).astype(BF16)


def kernel(inputs, support0, support1, W_ru_e, b_ru_e, W_c_e, b_c_e,
           W_ru_d, b_ru_d, W_c_d, b_c_d, W_proj, b_proj):
    x_all = inputs.reshape(L, B * N, L)

    out = pl.pallas_call(
        _fwd_kernel,
        out_shape=jax.ShapeDtypeStruct((HORIZON, B * N), F32),
    )(x_all, support0.T.astype(BF16), support1.T.astype(BF16),
      _reorder_w(W_ru_e, L + U, 2 * U), b_ru_e.reshape(1, 2 * U),
      _reorder_w(W_c_e, L + U, U), b_c_e.reshape(1, U),
      _reorder_w(W_ru_d, 1 + U, 2 * U), b_ru_d.reshape(1, 2 * U),
      _reorder_w(W_c_d, 1 + U, U), b_c_d.reshape(1, U),
      W_proj.T, b_proj.reshape(1, 1))

    return out.reshape(HORIZON, B, N)


# speedup vs baseline: 11.3888x; 1.4465x over previous
import jax
import jax.numpy as jnp
from jax.experimental import pallas as pl

N = 207
B = 16
L = 8
HORIZON = 8
U = 64
NUM_MAT = 5
F32 = jnp.float32
BF16 = jnp.bfloat16

_DNT = (((0,), (0,)), ((), ()))


def _fwd_kernel(x_all_ref, s0t_ref, s1t_ref,
                w0_ru_e_ref, w4_ru_e_ref, b_ru_e_ref,
                w0_c_e_ref, w4_c_e_ref, b_c_e_ref,
                w0_ru_d_ref, w4_ru_d_ref, b_ru_d_ref,
                w0_c_d_ref, w4_c_d_ref, b_c_d_ref,
                w_projt_ref, b_proj_ref,
                out_ref):
    s0t = s0t_ref[...]
    s1t = s1t_ref[...]

    def dotT(a, w):
        return jax.lax.dot_general(a, w, _DNT, preferred_element_type=F32)

    def gconv(cat, din, w0, w4, b):
        g0 = jnp.dot(cat, w0, preferred_element_type=F32)
        cat3 = cat.reshape(B, N, din)
        z1a = jnp.concatenate(
            [dotT(cat3[i], s0t).astype(BF16) for i in range(B)], axis=0)
        z1b = jnp.concatenate(
            [dotT(cat3[i], s1t).astype(BF16) for i in range(B)], axis=0)
        z2a = jnp.dot(z1a, s0t, preferred_element_type=F32).astype(BF16)
        z2b = jnp.dot(z1b, s1t, preferred_element_type=F32).astype(BF16)
        gates = []
        for i in range(B):
            sl = slice(i * din, (i + 1) * din)
            zcat = jnp.concatenate([z1a[sl], z2a[sl], z1b[sl], z2b[sl]], axis=0)
            gates.append(dotT(zcat, w4))
        return g0 + jnp.concatenate(gates, axis=0) + b

    def cell(x, h, din, w0_ru, w4_ru, b_ru, w0_c, w4_c, b_c):
        cat = jnp.concatenate([h, x], axis=1).astype(BF16)
        ru = jax.nn.sigmoid(gconv(cat, din, w0_ru, w4_ru, b_ru))
        r = ru[:, :U]
        u = ru[:, U:]
        cat2 = jnp.concatenate([r * h, x], axis=1).astype(BF16)
        c = jnp.tanh(gconv(cat2, din, w0_c, w4_c, b_c))
        return u * h + (1.0 - u) * c

    w0_ru_e = w0_ru_e_ref[...]
    w4_ru_e = w4_ru_e_ref[...]
    b_ru_e = b_ru_e_ref[...]
    w0_c_e = w0_c_e_ref[...]
    w4_c_e = w4_c_e_ref[...]
    b_c_e = b_c_e_ref[...]

    def enc_body(t, h):
        x = x_all_ref[pl.ds(t, 1)].reshape(B * N, L)
        return cell(x, h, U + L, w0_ru_e, w4_ru_e, b_ru_e,
                    w0_c_e, w4_c_e, b_c_e)

    h = jax.lax.fori_loop(0, L, enc_body, jnp.zeros((B * N, U), F32))

    w0_ru_d = w0_ru_d_ref[...]
    w4_ru_d = w4_ru_d_ref[...]
    b_ru_d = b_ru_d_ref[...]
    w0_c_d = w0_c_d_ref[...]
    w4_c_d = w4_c_d_ref[...]
    b_c_d = b_c_d_ref[...]
    w_projt = w_projt_ref[...]
    b_proj = b_proj_ref[0, 0]

    def dec_body(t, h):
        projt = jnp.dot(w_projt, h.T, preferred_element_type=F32) + b_proj
        xin = jnp.where(t == 0, jnp.zeros_like(projt), projt).T
        h2 = cell(xin, h, U + 1, w0_ru_d, w4_ru_d, b_ru_d,
                  w0_c_d, w4_c_d, b_c_d)
        proj2 = jnp.dot(w_projt, h2.T, preferred_element_type=F32) + b_proj
        out_ref[pl.ds(t, 1)] = proj2
        return h2

    jax.lax.fori_loop(0, HORIZON, dec_body, h)


def _prep_w(w, dx, dout):
    din = dx + U
    wm = w.reshape(din, NUM_MAT, dout)
    wm = jnp.concatenate([wm[dx:], wm[:dx]], axis=0)
    w0 = wm[:, 0] - wm[:, 2] - wm[:, 4]
    w4 = jnp.concatenate(
        [wm[:, 1], 2.0 * wm[:, 2], wm[:, 3], 2.0 * wm[:, 4]], axis=0)
    return w0.astype(BF16), w4.astype(BF16)


def kernel(inputs, support0, support1, W_ru_e, b_ru_e, W_c_e, b_c_e,
           W_ru_d, b_ru_d, W_c_d, b_c_d, W_proj, b_proj):
    x_all = inputs.reshape(L, B * N, L)

    w0_ru_e, w4_ru_e = _prep_w(W_ru_e, L, 2 * U)
    w0_c_e, w4_c_e = _prep_w(W_c_e, L, U)
    w0_ru_d, w4_ru_d = _prep_w(W_ru_d, 1, 2 * U)
    w0_c_d, w4_c_d = _prep_w(W_c_d, 1, U)

    out = pl.pallas_call(
        _fwd_kernel,
        out_shape=jax.ShapeDtypeStruct((HORIZON, B * N), F32),
    )(x_all, support0.T.astype(BF16), support1.T.astype(BF16),
      w0_ru_e, w4_ru_e, b_ru_e.reshape(1, 2 * U),
      w0_c_e, w4_c_e, b_c_e.reshape(1, U),
      w0_ru_d, w4_ru_d, b_ru_d.reshape(1, 2 * U),
      w0_c_d, w4_c_d, b_c_d.reshape(1, U),
      W_proj.T, b_proj.reshape(1, 1))

    return out.reshape(HORIZON, B, N)
